# Initial kernel scaffold; baseline (speedup 1.0000x reference)
import jax
import jax.numpy as jnp
from jax.experimental import pallas as pl


def kernel(positions, centroids, tables, W1, b1, W2, b2):
    raise NotImplementedError("write your pallas kernel here")


# speedup vs baseline: 3.4332x; 3.4332x over previous
import jax
import jax.numpy as jnp
import numpy as np
from jax import lax
from jax.experimental import pallas as pl
from jax.experimental.pallas import tpu as pltpu
from jax.experimental.pallas import tpu_sc as plsc

E = 8
L = 5
F = 2
T = 131072
H = 16
N = 131072
BASE_RES = 16
MAX_RES = 128
_B = float(np.exp((np.log(MAX_RES) - np.log(BASE_RES)) / (L - 1)))
_RES = [int(np.floor(BASE_RES * _B ** l)) for l in range(L)]
_P1 = np.int32(2654435761 - 2 ** 32)
_P2 = np.int32(805459861)

NC = 2
NS = 16
NW = NC * NS
PPW = N // NW
C = 128
NGRP = C // 16
NIDX = 8 * L * C


def _body(xs, ys, zs, cent, table8, w1, b1, w2, b2, out,
          xv, yv, zv, cv, w1v, b1v, w2v, b2v,
          idx_buf, qcol, wbuf, rows, ebuf, dens, sem):
    cid = lax.axis_index("c")
    sid = lax.axis_index("s")
    wid = sid * NC + cid

    pltpu.sync_copy(cent, cv)
    cxv = cv[0, pl.ds(0, 16)]
    cyv = cv[1, pl.ds(0, 16)]
    czv = cv[2, pl.ds(0, 16)]
    pltpu.sync_copy(w1, w1v)
    pltpu.sync_copy(b1, b1v)
    pltpu.sync_copy(w2, w2v)
    pltpu.sync_copy(b2, b2v)

    iota16 = lax.iota(jnp.int32, 16)

    @pl.loop(0, PPW // C)
    def _chunk(t):
        base = wid * PPW + t * C
        pltpu.sync_copy(xs.at[pl.ds(base, C)], xv)
        pltpu.sync_copy(ys.at[pl.ds(base, C)], yv)
        pltpu.sync_copy(zs.at[pl.ds(base, C)], zv)

        @pl.loop(0, NGRP)
        def _pass_a(g):
            off = g * 16
            px = xv[pl.ds(off, 16)]
            py = yv[pl.ds(off, 16)]
            pz = zv[pl.ds(off, 16)]
            beste = jnp.zeros((16,), jnp.int32)
            bestd = None
            for e in range(E):
                dx = px - cxv[e]
                dy = py - cyv[e]
                dz = pz - czv[e]
                d2 = dx * dx + dy * dy + dz * dz
                if e == 0:
                    bestd = d2
                else:
                    m = d2 < bestd
                    bestd = jnp.where(m, d2, bestd)
                    beste = jnp.where(m, e, beste)
            ebuf[pl.ds(off, 16)] = beste
            ebase = beste * (L * T)
            for l in range(L):
                res = float(_RES[l])
                fx = px * res
                fy = py * res
                fz = pz * res
                ix = fx.astype(jnp.int32)
                iy = fy.astype(jnp.int32)
                iz = fz.astype(jnp.int32)
                wx = fx - ix.astype(jnp.float32)
                wy = fy - iy.astype(jnp.float32)
                wz = fz - iz.astype(jnp.float32)
                hy0 = iy * _P1
                hz0 = iz * _P2
                hx = (ix, ix + 1)
                hy = (hy0, hy0 + _P1)
                hz = (hz0, hz0 + _P2)
                wxs = (1.0 - wx, wx)
                wys = (1.0 - wy, wy)
                wzs = (1.0 - wz, wz)
                for corner in range(8):
                    ox = corner & 1
                    oy = (corner >> 1) & 1
                    oz = (corner >> 2) & 1
                    h = (hx[ox] ^ hy[oy] ^ hz[oz]) & (T - 1)
                    gidx = ebase + (l * T) + h
                    cw = wxs[ox] * wys[oy] * wzs[oz]
                    p = (l * 8 + corner) * C + off
                    idx_buf[pl.ds(p, 16)] = jnp.right_shift(gidx, 2)
                    qcol[pl.ds(p, 16)] = jnp.left_shift(gidx & 3, 1)
                    wbuf[pl.ds(p, 16)] = cw

        cp = pltpu.make_async_copy(table8.at[idx_buf], rows, sem)
        cp.start()
        cp.wait()

        @pl.loop(0, NGRP)
        def _pass_b(g):
            off = g * 16
            e_vec = ebuf[pl.ds(off, 16)]
            enc = []
            for l in range(L):
                a0 = jnp.zeros((16,), jnp.float32)
                a1 = jnp.zeros((16,), jnp.float32)
                for corner in range(8):
                    p = (l * 8 + corner) * C + off
                    w = wbuf[pl.ds(p, 16)]
                    q2 = qcol[pl.ds(p, 16)]
                    ridx = p + iota16
                    f0 = plsc.load_gather(rows, [ridx, q2])
                    f1 = plsc.load_gather(rows, [ridx, q2 + 1])
                    a0 = a0 + w * f0
                    a1 = a1 + w * f1
                enc.append(a0)
                enc.append(a1)
            raw = plsc.load_gather(b2v, [e_vec])
            erow = e_vec * (2 * L)
            for hh in range(H):
                hsp = jnp.full((16,), hh, jnp.int32)
                a = plsc.load_gather(b1v, [e_vec, hsp])
                for f in range(2 * L):
                    wv = plsc.load_gather(w1v, [erow + f, hsp])
                    a = a + enc[f] * wv
                a = jnp.maximum(a, 0.0)
                w2g = plsc.load_gather(w2v, [e_vec, hsp])
                raw = raw + a * w2g
            dens[pl.ds(off, 16)] = jnp.exp(raw)

        pltpu.sync_copy(dens, out.at[pl.ds(base, C)])


_mesh = plsc.VectorSubcoreMesh(core_axis_name="c", subcore_axis_name="s")

_sc_kernel = pl.kernel(
    _body,
    out_type=jax.ShapeDtypeStruct((N,), jnp.float32),
    mesh=_mesh,
    compiler_params=pltpu.CompilerParams(
        needs_layout_passes=False, use_tc_tiling_on_sc=False
    ),
    scratch_types=[
        pltpu.VMEM((C,), jnp.float32),
        pltpu.VMEM((C,), jnp.float32),
        pltpu.VMEM((C,), jnp.float32),
        pltpu.VMEM((3, 16), jnp.float32),
        pltpu.VMEM((E * 2 * L, H), jnp.float32),
        pltpu.VMEM((E, H), jnp.float32),
        pltpu.VMEM((E, H), jnp.float32),
        pltpu.VMEM((E,), jnp.float32),
        pltpu.VMEM((NIDX,), jnp.int32),
        pltpu.VMEM((NIDX,), jnp.int32),
        pltpu.VMEM((NIDX,), jnp.float32),
        pltpu.VMEM((NIDX, 8), jnp.float32),
        pltpu.VMEM((C,), jnp.int32),
        pltpu.VMEM((C,), jnp.float32),
        pltpu.SemaphoreType.DMA,
    ],
)


def kernel(positions, centroids, tables, W1, b1, W2, b2):
    pos = positions.reshape(-1, 3)
    pos_t = pos.T
    xs = pos_t[0]
    ys = pos_t[1]
    zs = pos_t[2]
    table8 = tables.reshape(E * L * T // 4, 8)
    w1 = W1.reshape(E * 2 * L, H)
    w2 = W2[:, :, 0]
    b2f = b2[:, 0]
    cent_pad = jnp.pad(centroids.T, ((0, 0), (0, 16 - E)))
    dens = _sc_kernel(xs, ys, zs, cent_pad, table8, w1, b1, w2, b2f)
    return dens.reshape(positions.shape[:-1] + (1,))


# speedup vs baseline: 3.4389x; 1.0017x over previous
import jax
import jax.numpy as jnp
import numpy as np
from jax import lax
from jax.experimental import pallas as pl
from jax.experimental.pallas import tpu as pltpu
from jax.experimental.pallas import tpu_sc as plsc

E = 8
L = 5
F = 2
T = 131072
H = 16
N = 131072
BASE_RES = 16
MAX_RES = 128
_B = float(np.exp((np.log(MAX_RES) - np.log(BASE_RES)) / (L - 1)))
_RES = [int(np.floor(BASE_RES * _B ** l)) for l in range(L)]
_P1 = np.int32(2654435761 - 2 ** 32)
_P2 = np.int32(805459861)

NC = 2
NS = 16
NW = NC * NS
PPW = N // NW
C = 128
NGRP = C // 16
NIDX = 8 * L * C


def _body(pos, cent, table8, w1, b1, w2, b2, out,
          posv, cv, w1v, b1v, w2v, b2v,
          idx_buf, qcol, wbuf, rows, ebuf, dens, sem):
    cid = lax.axis_index("c")
    sid = lax.axis_index("s")
    wid = sid * NC + cid

    pltpu.sync_copy(cent, cv)
    cxv = cv[0, pl.ds(0, 16)]
    cyv = cv[1, pl.ds(0, 16)]
    czv = cv[2, pl.ds(0, 16)]
    pltpu.sync_copy(w1, w1v)
    pltpu.sync_copy(b1, b1v)
    pltpu.sync_copy(w2, w2v)
    pltpu.sync_copy(b2, b2v)

    iota16 = lax.iota(jnp.int32, 16)
    zero16v = jnp.zeros((16,), jnp.int32)
    one16v = jnp.ones((16,), jnp.int32)
    two16v = jnp.full((16,), 2, jnp.int32)

    @pl.loop(0, PPW // C)
    def _chunk(t):
        base = wid * PPW + t * C
        pltpu.sync_copy(pos.at[pl.ds(base, C), :], posv)

        @pl.loop(0, NGRP)
        def _pass_a(g):
            off = g * 16
            oi = off + iota16
            px = plsc.load_gather(posv, [oi, zero16v])
            py = plsc.load_gather(posv, [oi, one16v])
            pz = plsc.load_gather(posv, [oi, two16v])
            beste = jnp.zeros((16,), jnp.int32)
            bestd = None
            for e in range(E):
                dx = px - cxv[e]
                dy = py - cyv[e]
                dz = pz - czv[e]
                d2 = dx * dx + dy * dy + dz * dz
                if e == 0:
                    bestd = d2
                else:
                    m = d2 < bestd
                    bestd = jnp.where(m, d2, bestd)
                    beste = jnp.where(m, e, beste)
            ebuf[pl.ds(off, 16)] = beste
            ebase = beste * (L * T)
            for l in range(L):
                res = float(_RES[l])
                fx = px * res
                fy = py * res
                fz = pz * res
                ix = fx.astype(jnp.int32)
                iy = fy.astype(jnp.int32)
                iz = fz.astype(jnp.int32)
                wx = fx - ix.astype(jnp.float32)
                wy = fy - iy.astype(jnp.float32)
                wz = fz - iz.astype(jnp.float32)
                hy0 = iy * _P1
                hz0 = iz * _P2
                hx = (ix, ix + 1)
                hy = (hy0, hy0 + _P1)
                hz = (hz0, hz0 + _P2)
                wxs = (1.0 - wx, wx)
                wys = (1.0 - wy, wy)
                wzs = (1.0 - wz, wz)
                for corner in range(8):
                    ox = corner & 1
                    oy = (corner >> 1) & 1
                    oz = (corner >> 2) & 1
                    h = (hx[ox] ^ hy[oy] ^ hz[oz]) & (T - 1)
                    gidx = ebase + (l * T) + h
                    cw = wxs[ox] * wys[oy] * wzs[oz]
                    p = (l * 8 + corner) * C + off
                    idx_buf[pl.ds(p, 16)] = jnp.right_shift(gidx, 2)
                    qcol[pl.ds(p, 16)] = jnp.left_shift(gidx & 3, 1)
                    wbuf[pl.ds(p, 16)] = cw

        cp = pltpu.make_async_copy(table8.at[idx_buf], rows, sem)
        cp.start()
        cp.wait()

        @pl.loop(0, NGRP)
        def _pass_b(g):
            off = g * 16
            e_vec = ebuf[pl.ds(off, 16)]
            enc = []
            for l in range(L):
                a0 = jnp.zeros((16,), jnp.float32)
                a1 = jnp.zeros((16,), jnp.float32)
                for corner in range(8):
                    p = (l * 8 + corner) * C + off
                    w = wbuf[pl.ds(p, 16)]
                    q2 = qcol[pl.ds(p, 16)]
                    ridx = p + iota16
                    f0 = plsc.load_gather(rows, [ridx, q2])
                    f1 = plsc.load_gather(rows, [ridx, q2 + 1])
                    a0 = a0 + w * f0
                    a1 = a1 + w * f1
                enc.append(a0)
                enc.append(a1)
            raw = plsc.load_gather(b2v, [e_vec])
            erow = e_vec * (2 * L)
            for hh in range(H):
                hsp = jnp.full((16,), hh, jnp.int32)
                a = plsc.load_gather(b1v, [e_vec, hsp])
                for f in range(2 * L):
                    wv = plsc.load_gather(w1v, [erow + f, hsp])
                    a = a + enc[f] * wv
                a = jnp.maximum(a, 0.0)
                w2g = plsc.load_gather(w2v, [e_vec, hsp])
                raw = raw + a * w2g
            dens[pl.ds(off, 16)] = jnp.exp(raw)

        pltpu.sync_copy(dens, out.at[pl.ds(base, C)])


_mesh = plsc.VectorSubcoreMesh(core_axis_name="c", subcore_axis_name="s")

_sc_kernel = pl.kernel(
    _body,
    out_type=jax.ShapeDtypeStruct((N,), jnp.float32),
    mesh=_mesh,
    compiler_params=pltpu.CompilerParams(
        needs_layout_passes=False, use_tc_tiling_on_sc=False
    ),
    scratch_types=[
        pltpu.VMEM((C, 3), jnp.float32),
        pltpu.VMEM((3, 16), jnp.float32),
        pltpu.VMEM((E * 2 * L, H), jnp.float32),
        pltpu.VMEM((E, H), jnp.float32),
        pltpu.VMEM((E, H), jnp.float32),
        pltpu.VMEM((E,), jnp.float32),
        pltpu.VMEM((NIDX,), jnp.int32),
        pltpu.VMEM((NIDX,), jnp.int32),
        pltpu.VMEM((NIDX,), jnp.float32),
        pltpu.VMEM((NIDX, 8), jnp.float32),
        pltpu.VMEM((C,), jnp.int32),
        pltpu.VMEM((C,), jnp.float32),
        pltpu.SemaphoreType.DMA,
    ],
)


def kernel(positions, centroids, tables, W1, b1, W2, b2):
    pos = positions.reshape(-1, 3)
    table8 = tables.reshape(E * L * T // 4, 8)
    w1 = W1.reshape(E * 2 * L, H)
    w2 = W2[:, :, 0]
    b2f = b2[:, 0]
    cent_pad = jnp.pad(centroids.T, ((0, 0), (0, 16 - E)))
    dens = _sc_kernel(pos, cent_pad, table8, w1, b1, w2, b2f)
    return dens.reshape(positions.shape[:-1] + (1,))


# speedup vs baseline: 27.6604x; 8.0433x over previous
import jax
import jax.numpy as jnp
import numpy as np
from jax import lax
from jax.experimental import pallas as pl
from jax.experimental.pallas import tpu as pltpu
from jax.experimental.pallas import tpu_sc as plsc

E = 8
L = 5
F = 2
T = 131072
H = 16
N = 131072
BASE_RES = 16
MAX_RES = 128
_B = float(np.exp((np.log(MAX_RES) - np.log(BASE_RES)) / (L - 1)))
_RES = [int(np.floor(BASE_RES * _B ** l)) for l in range(L)]
_P1 = np.int32(2654435761 - 2 ** 32)
_P2 = np.int32(805459861)

NC = 2
NS = 16
NW = NC * NS
PPW = N // NW
C = 256
NGRP = C // 16
NIDX = 8 * L * C


def _body(pos, cent, tflat, w1, b1, w2, b2, out,
          posv, cv, w1v, b1v, w2v, b2v,
          idx_buf, wbuf, rows, ebuf, dens, sem):
    cid = lax.axis_index("c")
    sid = lax.axis_index("s")
    wid = sid * NC + cid

    pltpu.sync_copy(cent, cv)
    cxv = cv[0, pl.ds(0, 16)]
    cyv = cv[1, pl.ds(0, 16)]
    czv = cv[2, pl.ds(0, 16)]
    pltpu.sync_copy(w1, w1v)
    pltpu.sync_copy(b1, b1v)
    pltpu.sync_copy(w2, w2v)
    pltpu.sync_copy(b2, b2v)

    iota16 = lax.iota(jnp.int32, 16)
    zero16v = jnp.zeros((16,), jnp.int32)
    one16v = jnp.ones((16,), jnp.int32)
    two16v = jnp.full((16,), 2, jnp.int32)

    @pl.loop(0, PPW // C)
    def _chunk(t):
        base = wid * PPW + t * C
        pltpu.sync_copy(pos.at[pl.ds(base, C), :], posv)

        @pl.loop(0, NGRP)
        def _pass_a(g):
            off = g * 16
            oi = off + iota16
            px = plsc.load_gather(posv, [oi, zero16v])
            py = plsc.load_gather(posv, [oi, one16v])
            pz = plsc.load_gather(posv, [oi, two16v])
            beste = jnp.zeros((16,), jnp.int32)
            bestd = None
            for e in range(E):
                dx = px - cxv[e]
                dy = py - cyv[e]
                dz = pz - czv[e]
                d2 = dx * dx + dy * dy + dz * dz
                if e == 0:
                    bestd = d2
                else:
                    m = d2 < bestd
                    bestd = jnp.where(m, d2, bestd)
                    beste = jnp.where(m, e, beste)
            ebuf[pl.ds(off, 16)] = beste
            ebase = beste * (L * T * F)
            for l in range(L):
                res = float(_RES[l])
                fx = px * res
                fy = py * res
                fz = pz * res
                ix = fx.astype(jnp.int32)
                iy = fy.astype(jnp.int32)
                iz = fz.astype(jnp.int32)
                wx = fx - ix.astype(jnp.float32)
                wy = fy - iy.astype(jnp.float32)
                wz = fz - iz.astype(jnp.float32)
                hy0 = iy * _P1
                hz0 = iz * _P2
                hx = (ix, ix + 1)
                hy = (hy0, hy0 + _P1)
                hz = (hz0, hz0 + _P2)
                wxs = (1.0 - wx, wx)
                wys = (1.0 - wy, wy)
                wzs = (1.0 - wz, wz)
                for corner in range(8):
                    ox = corner & 1
                    oy = (corner >> 1) & 1
                    oz = (corner >> 2) & 1
                    h = (hx[ox] ^ hy[oy] ^ hz[oz]) & (T - 1)
                    w0 = ebase + (l * (T * F)) \
                        + jnp.left_shift(jnp.right_shift(h, 7), 8) + (h & 127)
                    cw = wxs[ox] * wys[oy] * wzs[oz]
                    p = (l * 8 + corner) * C + off
                    idx_buf[pl.ds(p, 16)] = w0
                    idx_buf[pl.ds(NIDX + p, 16)] = w0 + 128
                    wbuf[pl.ds(p, 16)] = cw

        cp = pltpu.make_async_copy(tflat.at[idx_buf], rows, sem)
        cp.start()
        cp.wait()

        @pl.loop(0, NGRP)
        def _pass_b(g):
            off = g * 16
            e_vec = ebuf[pl.ds(off, 16)]
            enc = []
            for l in range(L):
                a0 = jnp.zeros((16,), jnp.float32)
                a1 = jnp.zeros((16,), jnp.float32)
                for corner in range(8):
                    p = (l * 8 + corner) * C + off
                    w = wbuf[pl.ds(p, 16)]
                    f0 = rows[pl.ds(p, 16)]
                    f1 = rows[pl.ds(NIDX + p, 16)]
                    a0 = a0 + w * f0
                    a1 = a1 + w * f1
                enc.append(a0)
                enc.append(a1)
            raw = plsc.load_gather(b2v, [e_vec])
            erow = e_vec * (2 * L)
            for hh in range(H):
                hsp = jnp.full((16,), hh, jnp.int32)
                a = plsc.load_gather(b1v, [e_vec, hsp])
                for f in range(2 * L):
                    wv = plsc.load_gather(w1v, [erow + f, hsp])
                    a = a + enc[f] * wv
                a = jnp.maximum(a, 0.0)
                w2g = plsc.load_gather(w2v, [e_vec, hsp])
                raw = raw + a * w2g
            dens[pl.ds(off, 16)] = jnp.exp(raw)

        pltpu.sync_copy(dens, out.at[pl.ds(base, C)])


_mesh = plsc.VectorSubcoreMesh(core_axis_name="c", subcore_axis_name="s")

_sc_kernel = pl.kernel(
    _body,
    out_type=jax.ShapeDtypeStruct((N,), jnp.float32),
    mesh=_mesh,
    compiler_params=pltpu.CompilerParams(
        needs_layout_passes=False, use_tc_tiling_on_sc=False
    ),
    scratch_types=[
        pltpu.VMEM((C, 3), jnp.float32),
        pltpu.VMEM((3, 16), jnp.float32),
        pltpu.VMEM((E * 2 * L, H), jnp.float32),
        pltpu.VMEM((E, H), jnp.float32),
        pltpu.VMEM((E, H), jnp.float32),
        pltpu.VMEM((E,), jnp.float32),
        pltpu.VMEM((2 * NIDX,), jnp.int32),
        pltpu.VMEM((NIDX,), jnp.float32),
        pltpu.VMEM((2 * NIDX,), jnp.float32),
        pltpu.VMEM((C,), jnp.int32),
        pltpu.VMEM((C,), jnp.float32),
        pltpu.SemaphoreType.DMA,
    ],
)


def kernel(positions, centroids, tables, W1, b1, W2, b2):
    pos = positions.reshape(-1, 3)
    tflat = jnp.transpose(
        tables.reshape(E, L, T // 128, 128, F), (0, 1, 2, 4, 3)
    ).reshape(E * L * T * F)
    w1 = W1.reshape(E * 2 * L, H)
    w2 = W2[:, :, 0]
    b2f = b2[:, 0]
    cent_pad = jnp.pad(centroids.T, ((0, 0), (0, 16 - E)))
    dens = _sc_kernel(pos, cent_pad, tflat, w1, b1, w2, b2f)
    return dens.reshape(positions.shape[:-1] + (1,))


# speedup vs baseline: 41.7066x; 1.5078x over previous
import jax
import jax.numpy as jnp
import numpy as np
from jax import lax
from jax.experimental import pallas as pl
from jax.experimental.pallas import tpu as pltpu
from jax.experimental.pallas import tpu_sc as plsc

E = 8
L = 5
F = 2
T = 131072
H = 16
N = 131072
BASE_RES = 16
MAX_RES = 128
_B = float(np.exp((np.log(MAX_RES) - np.log(BASE_RES)) / (L - 1)))
_RES = [int(np.floor(BASE_RES * _B ** l)) for l in range(L)]
_P1 = np.int32(2654435761 - 2 ** 32)
_P2 = np.int32(805459861)

NC = 2
NS = 16
NW = NC * NS
PPW = N // NW
C = 256
NGRP = C // 16
NCHUNK = PPW // C
NIDX = 8 * L * C


def _body(pos, cent, tflat, w1, b1, w2, b2, out,
          posv, cv, w1v, b1v, w2v, b2v,
          idx_buf, wbuf, rows, ebuf, dens, sems):
    cid = lax.axis_index("c")
    sid = lax.axis_index("s")
    wid = sid * NC + cid

    pltpu.sync_copy(cent, cv)
    cxv = cv[0, pl.ds(0, 16)]
    cyv = cv[1, pl.ds(0, 16)]
    czv = cv[2, pl.ds(0, 16)]
    pltpu.sync_copy(w1, w1v)
    pltpu.sync_copy(b1, b1v)
    pltpu.sync_copy(w2, w2v)
    pltpu.sync_copy(b2, b2v)

    iota16 = lax.iota(jnp.int32, 16)
    zero16v = jnp.zeros((16,), jnp.int32)
    one16v = jnp.ones((16,), jnp.int32)
    two16v = jnp.full((16,), 2, jnp.int32)

    def pass_a(t, par):
        base = wid * PPW + t * C
        pltpu.sync_copy(pos.at[pl.ds(base, C), :], posv.at[pl.ds(par * C, C), :])

        @pl.loop(0, NGRP)
        def _pa(g):
            off = g * 16
            oi = par * C + off + iota16
            px = plsc.load_gather(posv, [oi, zero16v])
            py = plsc.load_gather(posv, [oi, one16v])
            pz = plsc.load_gather(posv, [oi, two16v])
            beste = jnp.zeros((16,), jnp.int32)
            bestd = None
            for e in range(E):
                dx = px - cxv[e]
                dy = py - cyv[e]
                dz = pz - czv[e]
                d2 = dx * dx + dy * dy + dz * dz
                if e == 0:
                    bestd = d2
                else:
                    m = d2 < bestd
                    bestd = jnp.where(m, d2, bestd)
                    beste = jnp.where(m, e, beste)
            ebuf[par, pl.ds(off, 16)] = beste
            ebase = beste * (L * T * F)
            for l in range(L):
                res = float(_RES[l])
                fx = px * res
                fy = py * res
                fz = pz * res
                ix = fx.astype(jnp.int32)
                iy = fy.astype(jnp.int32)
                iz = fz.astype(jnp.int32)
                wx = fx - ix.astype(jnp.float32)
                wy = fy - iy.astype(jnp.float32)
                wz = fz - iz.astype(jnp.float32)
                hy0 = iy * _P1
                hz0 = iz * _P2
                hx = (ix, ix + 1)
                hy = (hy0, hy0 + _P1)
                hz = (hz0, hz0 + _P2)
                wxs = (1.0 - wx, wx)
                wys = (1.0 - wy, wy)
                wzs = (1.0 - wz, wz)
                for corner in range(8):
                    ox = corner & 1
                    oy = (corner >> 1) & 1
                    oz = (corner >> 2) & 1
                    h = (hx[ox] ^ hy[oy] ^ hz[oz]) & (T - 1)
                    w0 = ebase + (l * (T * F)) \
                        + jnp.left_shift(jnp.right_shift(h, 7), 8) + (h & 127)
                    cw = wxs[ox] * wys[oy] * wzs[oz]
                    p = (l * 8 + corner) * C + off
                    idx_buf[par, pl.ds(p, 16)] = w0
                    idx_buf[par, pl.ds(NIDX + p, 16)] = w0 + 128
                    wbuf[par, pl.ds(p, 16)] = cw

    def fire(par):
        pltpu.make_async_copy(
            tflat.at[idx_buf.at[par]], rows.at[par], sems.at[par]
        ).start()

    def wait(par):
        pltpu.make_async_copy(
            tflat.at[idx_buf.at[par]], rows.at[par], sems.at[par]
        ).wait()

    def pass_b(t, par):
        base = wid * PPW + t * C

        @pl.loop(0, NGRP)
        def _pb(g):
            off = g * 16
            e_vec = ebuf[par, pl.ds(off, 16)]
            enc = []
            for l in range(L):
                a0 = jnp.zeros((16,), jnp.float32)
                a1 = jnp.zeros((16,), jnp.float32)
                for corner in range(8):
                    p = (l * 8 + corner) * C + off
                    w = wbuf[par, pl.ds(p, 16)]
                    f0 = rows[par, pl.ds(p, 16)]
                    f1 = rows[par, pl.ds(NIDX + p, 16)]
                    a0 = a0 + w * f0
                    a1 = a1 + w * f1
                enc.append(a0)
                enc.append(a1)
            raw = plsc.load_gather(b2v, [e_vec])
            erow = e_vec * (2 * L)
            for hh in range(H):
                hsp = jnp.full((16,), hh, jnp.int32)
                a = plsc.load_gather(b1v, [e_vec, hsp])
                for f in range(2 * L):
                    wv = plsc.load_gather(w1v, [erow + f, hsp])
                    a = a + enc[f] * wv
                a = jnp.maximum(a, 0.0)
                w2g = plsc.load_gather(w2v, [e_vec, hsp])
                raw = raw + a * w2g
            dens[par, pl.ds(off, 16)] = jnp.exp(raw)

        pltpu.sync_copy(dens.at[par], out.at[pl.ds(base, C)])

    pass_a(0, 0)
    fire(0)

    @pl.loop(0, NCHUNK - 1)
    def _pipe(t):
        par = t & 1
        q = 1 - par
        pass_a(t + 1, q)
        fire(q)
        wait(par)
        pass_b(t, par)

    wait((NCHUNK - 1) & 1)
    pass_b(NCHUNK - 1, (NCHUNK - 1) & 1)


_mesh = plsc.VectorSubcoreMesh(core_axis_name="c", subcore_axis_name="s")

_sc_kernel = pl.kernel(
    _body,
    out_type=jax.ShapeDtypeStruct((N,), jnp.float32),
    mesh=_mesh,
    compiler_params=pltpu.CompilerParams(
        needs_layout_passes=False, use_tc_tiling_on_sc=False
    ),
    scratch_types=[
        pltpu.VMEM((2 * C, 3), jnp.float32),
        pltpu.VMEM((3, 16), jnp.float32),
        pltpu.VMEM((E * 2 * L, H), jnp.float32),
        pltpu.VMEM((E, H), jnp.float32),
        pltpu.VMEM((E, H), jnp.float32),
        pltpu.VMEM((E,), jnp.float32),
        pltpu.VMEM((2, 2 * NIDX), jnp.int32),
        pltpu.VMEM((2, NIDX), jnp.float32),
        pltpu.VMEM((2, 2 * NIDX), jnp.float32),
        pltpu.VMEM((2, C), jnp.int32),
        pltpu.VMEM((2, C), jnp.float32),
        pltpu.SemaphoreType.DMA((2,)),
    ],
)


def kernel(positions, centroids, tables, W1, b1, W2, b2):
    pos = positions.reshape(-1, 3)
    tflat = jnp.transpose(
        tables.reshape(E, L, T // 128, 128, F), (0, 1, 2, 4, 3)
    ).reshape(E * L * T * F)
    w1 = W1.reshape(E * 2 * L, H)
    w2 = W2[:, :, 0]
    b2f = b2[:, 0]
    cent_pad = jnp.pad(centroids.T, ((0, 0), (0, 16 - E)))
    dens = _sc_kernel(pos, cent_pad, tflat, w1, b1, w2, b2f)
    return dens.reshape(positions.shape[:-1] + (1,))


# speedup vs baseline: 50.8115x; 1.2183x over previous
import jax
import jax.numpy as jnp
import numpy as np
from jax import lax
from jax.experimental import pallas as pl
from jax.experimental.pallas import tpu as pltpu
from jax.experimental.pallas import tpu_sc as plsc

E = 8
L = 5
F = 2
T = 131072
H = 16
N = 131072
BASE_RES = 16
MAX_RES = 128
_B = float(np.exp((np.log(MAX_RES) - np.log(BASE_RES)) / (L - 1)))
_RES = [int(np.floor(BASE_RES * _B ** l)) for l in range(L)]
_P1 = np.int32(2654435761 - 2 ** 32)
_P2 = np.int32(805459861)

NC = 2
NS = 16
NW = NC * NS
PPW = N // NW
C = 256
NGRP = C // 16
NCHUNK = PPW // C
NIDX = 8 * L * C


def _body(xs, ys, zs, cent, tflat, w1, b1, w2, b2, out,
          xv, yv, zv, cv, w1v, b1v, w2v, b2v,
          idx_buf, wbuf, rows, ebuf, dens, sems):
    cid = lax.axis_index("c")
    sid = lax.axis_index("s")
    wid = sid * NC + cid

    pltpu.sync_copy(cent, cv)
    cxv = cv[0, pl.ds(0, 16)]
    cyv = cv[1, pl.ds(0, 16)]
    czv = cv[2, pl.ds(0, 16)]
    pltpu.sync_copy(w1, w1v)
    pltpu.sync_copy(b1, b1v)
    pltpu.sync_copy(w2, w2v)
    pltpu.sync_copy(b2, b2v)

    iota16 = lax.iota(jnp.int32, 16)
    zero16v = jnp.zeros((16,), jnp.int32)
    one16v = jnp.ones((16,), jnp.int32)
    two16v = jnp.full((16,), 2, jnp.int32)

    def pass_a(t, par):
        base = wid * PPW + t * C
        pltpu.sync_copy(xs.at[pl.ds(base, C)], xv.at[pl.ds(par * C, C)])
        pltpu.sync_copy(ys.at[pl.ds(base, C)], yv.at[pl.ds(par * C, C)])
        pltpu.sync_copy(zs.at[pl.ds(base, C)], zv.at[pl.ds(par * C, C)])

        @pl.loop(0, NGRP)
        def _pa(g):
            off = g * 16
            po = par * C + off
            px = xv[pl.ds(po, 16)]
            py = yv[pl.ds(po, 16)]
            pz = zv[pl.ds(po, 16)]
            beste = jnp.zeros((16,), jnp.int32)
            bestd = None
            for e in range(E):
                dx = px - cxv[e]
                dy = py - cyv[e]
                dz = pz - czv[e]
                d2 = dx * dx + dy * dy + dz * dz
                if e == 0:
                    bestd = d2
                else:
                    m = d2 < bestd
                    bestd = jnp.where(m, d2, bestd)
                    beste = jnp.where(m, e, beste)
            ebuf[par, pl.ds(off, 16)] = beste
            ebase = beste * (L * T * F)
            for l in range(L):
                res = float(_RES[l])
                fx = px * res
                fy = py * res
                fz = pz * res
                ix = fx.astype(jnp.int32)
                iy = fy.astype(jnp.int32)
                iz = fz.astype(jnp.int32)
                wx = fx - ix.astype(jnp.float32)
                wy = fy - iy.astype(jnp.float32)
                wz = fz - iz.astype(jnp.float32)
                hy0 = iy * _P1
                hz0 = iz * _P2
                hx = (ix, ix + 1)
                hy = (hy0, hy0 + _P1)
                hz = (hz0, hz0 + _P2)
                wxs = (1.0 - wx, wx)
                wys = (1.0 - wy, wy)
                wzs = (1.0 - wz, wz)
                for corner in range(8):
                    ox = corner & 1
                    oy = (corner >> 1) & 1
                    oz = (corner >> 2) & 1
                    h = (hx[ox] ^ hy[oy] ^ hz[oz]) & (T - 1)
                    w0 = ebase + (l * (T * F)) \
                        + jnp.left_shift(jnp.right_shift(h, 7), 8) + (h & 127)
                    cw = wxs[ox] * wys[oy] * wzs[oz]
                    p = (l * 8 + corner) * C + off
                    idx_buf[par, pl.ds(p, 16)] = w0
                    idx_buf[par, pl.ds(NIDX + p, 16)] = w0 + 128
                    wbuf[par, pl.ds(p, 16)] = cw

    def fire(par):
        pltpu.make_async_copy(
            tflat.at[idx_buf.at[par]], rows.at[par], sems.at[par]
        ).start()

    def wait(par):
        pltpu.make_async_copy(
            tflat.at[idx_buf.at[par]], rows.at[par], sems.at[par]
        ).wait()

    def pass_b(t, par):
        base = wid * PPW + t * C

        @pl.loop(0, NGRP)
        def _pb(g):
            off = g * 16
            e_vec = ebuf[par, pl.ds(off, 16)]
            enc = []
            for l in range(L):
                a0 = jnp.zeros((16,), jnp.float32)
                a1 = jnp.zeros((16,), jnp.float32)
                for corner in range(8):
                    p = (l * 8 + corner) * C + off
                    w = wbuf[par, pl.ds(p, 16)]
                    f0 = rows[par, pl.ds(p, 16)]
                    f1 = rows[par, pl.ds(NIDX + p, 16)]
                    a0 = a0 + w * f0
                    a1 = a1 + w * f1
                enc.append(a0)
                enc.append(a1)
            raw = plsc.load_gather(b2v, [e_vec])
            erow = e_vec * (2 * L)
            for hh in range(H):
                hsp = jnp.full((16,), hh, jnp.int32)
                a = plsc.load_gather(b1v, [e_vec, hsp])
                for f in range(2 * L):
                    wv = plsc.load_gather(w1v, [erow + f, hsp])
                    a = a + enc[f] * wv
                a = jnp.maximum(a, 0.0)
                w2g = plsc.load_gather(w2v, [e_vec, hsp])
                raw = raw + a * w2g
            dens[par, pl.ds(off, 16)] = jnp.exp(raw)

        pltpu.sync_copy(dens.at[par], out.at[pl.ds(base, C)])

    pass_a(0, 0)
    fire(0)

    @pl.loop(0, NCHUNK - 1)
    def _pipe(t):
        par = t & 1
        q = 1 - par
        pass_a(t + 1, q)
        fire(q)
        wait(par)
        pass_b(t, par)

    wait((NCHUNK - 1) & 1)
    pass_b(NCHUNK - 1, (NCHUNK - 1) & 1)


_mesh = plsc.VectorSubcoreMesh(core_axis_name="c", subcore_axis_name="s")

_sc_kernel = pl.kernel(
    _body,
    out_type=jax.ShapeDtypeStruct((N,), jnp.float32),
    mesh=_mesh,
    compiler_params=pltpu.CompilerParams(
        needs_layout_passes=False, use_tc_tiling_on_sc=False
    ),
    scratch_types=[
        pltpu.VMEM((2 * C,), jnp.float32),
        pltpu.VMEM((2 * C,), jnp.float32),
        pltpu.VMEM((2 * C,), jnp.float32),
        pltpu.VMEM((3, 16), jnp.float32),
        pltpu.VMEM((E * 2 * L, H), jnp.float32),
        pltpu.VMEM((E, H), jnp.float32),
        pltpu.VMEM((E, H), jnp.float32),
        pltpu.VMEM((E,), jnp.float32),
        pltpu.VMEM((2, 2 * NIDX), jnp.int32),
        pltpu.VMEM((2, NIDX), jnp.float32),
        pltpu.VMEM((2, 2 * NIDX), jnp.float32),
        pltpu.VMEM((2, C), jnp.int32),
        pltpu.VMEM((2, C), jnp.float32),
        pltpu.SemaphoreType.DMA((2,)),
    ],
)


def kernel(positions, centroids, tables, W1, b1, W2, b2):
    pos_t = positions.reshape(-1, 3).T
    xs = pos_t[0]
    ys = pos_t[1]
    zs = pos_t[2]
    tflat = jnp.transpose(
        tables.reshape(E, L, T // 128, 128, F), (0, 1, 2, 4, 3)
    ).reshape(E * L * T * F)
    w1 = W1.reshape(E * 2 * L, H)
    w2 = W2[:, :, 0]
    b2f = b2[:, 0]
    cent_pad = jnp.pad(centroids.T, ((0, 0), (0, 16 - E)))
    dens = _sc_kernel(xs, ys, zs, cent_pad, tflat, w1, b1, w2, b2f)
    return dens.reshape(positions.shape[:-1] + (1,))
